# baseline (speedup 1.0000x reference)
import functools

import jax
import jax.numpy as jnp
from jax import lax
from jax.experimental import pallas as pl
from jax.experimental.pallas import tpu as pltpu
from jax.experimental.pallas import tpu_sc as plsc

NC = 2
NS = 16
LANES = 16
CHUNK = 128
PHASES = 2


def _scale_kernel(ego_ref, norm_ref, g_ref):
    g_ref[...] = ego_ref[...] * norm_ref[...]


def _scale(ego_p, norm_p):
    npad, d = ego_p.shape
    block = npad // 16
    grid = 16
    return pl.pallas_call(
        _scale_kernel,
        grid=(grid,),
        in_specs=[
            pl.BlockSpec((block, d), lambda i: (i, 0)),
            pl.BlockSpec((block, 1), lambda i: (i, 0)),
        ],
        out_specs=pl.BlockSpec((block, d), lambda i: (i, 0)),
        out_shape=jax.ShapeDtypeStruct((npad, d), jnp.float32),
    )(ego_p, norm_p)


def _make_segsum(npad, d, steps0, steps1):
    rows_per_tile = npad // NS
    zcopies = rows_per_tile // CHUNK
    zrem = rows_per_tile % CHUNK
    hs_max = max(steps0, steps1) // PHASES
    mesh = plsc.VectorSubcoreMesh(core_axis_name="c", subcore_axis_name="s")

    @functools.partial(
        pl.kernel,
        out_type=jax.ShapeDtypeStruct((NC, npad, d), jnp.float32),
        mesh=mesh,
        scratch_types=[
            pltpu.VMEM((hs_max, CHUNK), jnp.int32),
            pltpu.VMEM((hs_max, CHUNK), jnp.int32),
            pltpu.VMEM((2, CHUNK, d), jnp.float32),
            pltpu.VMEM_SHARED((npad, d), jnp.float32),
            pltpu.SemaphoreType.DMA,
            pltpu.SemaphoreType.DMA,
        ],
    )
    def segsum(g_hbm, src0_hbm, dst0_hbm, src1_hbm, dst1_hbm, out_hbm,
               src_all, dst_all, rows, acc_sh, sem0, sem1):
        c = lax.axis_index("c")
        s = lax.axis_index("s")
        sems = (sem0, sem1)

        def zrow(i, _):
            def zcol(j, _):
                rows[0, i, pl.ds(j * LANES, LANES)] = jnp.zeros((LANES,), jnp.float32)
                return 0
            return lax.fori_loop(0, d // LANES, zcol, 0)
        lax.fori_loop(0, CHUNK, zrow, 0)

        zbase = s * rows_per_tile
        for k in range(zcopies):
            pltpu.sync_copy(rows.at[0], acc_sh.at[pl.ds(zbase + k * CHUNK, CHUNK)])
        if zrem:
            pltpu.sync_copy(
                rows.at[0, pl.ds(0, zrem)],
                acc_sh.at[pl.ds(zbase + zcopies * CHUNK, zrem)],
            )

        plsc.subcore_barrier()

        def gather(t, b):
            pltpu.async_copy(g_hbm.at[src_all.at[t]], rows.at[b], sems[b])

        def wait_gather(t, b):
            pltpu.make_async_copy(g_hbm.at[src_all.at[t]], rows.at[b], sems[b]).wait()

        def scat(t, b):
            pltpu.sync_copy(rows.at[b], acc_sh.at[dst_all.at[t]], add=True)

        def pair(p, _):
            for b in range(2):
                t = 2 * p + b
                wait_gather(t, b)
                scat(t, b)
                gather(t + 2, b)
            return 0

        def run_core(src_hbm, dst_hbm, hsteps):
            for ph in range(PHASES):
                pltpu.sync_copy(src_hbm.at[s, pl.ds(ph * hsteps, hsteps)], src_all.at[pl.ds(0, hsteps)])
                pltpu.sync_copy(dst_hbm.at[s, pl.ds(ph * hsteps, hsteps)], dst_all.at[pl.ds(0, hsteps)])
                gather(0, 0)
                gather(1, 1)
                lax.fori_loop(0, hsteps // 2 - 1, pair, 0)
                for b in range(2):
                    t = hsteps - 2 + b
                    wait_gather(t, b)
                    scat(t, b)

        @pl.when(c == 0)
        def _():
            run_core(src0_hbm, dst0_hbm, steps0 // PHASES)

        @pl.when(c == 1)
        def _():
            run_core(src1_hbm, dst1_hbm, steps1 // PHASES)

        plsc.subcore_barrier()
        pltpu.sync_copy(
            acc_sh.at[pl.ds(zbase, rows_per_tile)],
            out_hbm.at[c, pl.ds(zbase, rows_per_tile)],
        )

    return segsum


def _finish_kernel(sa_ref, sb_ref, ego_ref, norm_ref, w1_ref, w2_ref, out_ref):
    s = sa_ref[...] + sb_ref[...]
    ego = ego_ref[...]
    nrm = norm_ref[...]
    t1 = ego + nrm * s
    t2 = (nrm * ego) * s
    h = jnp.dot(t1, w1_ref[...], preferred_element_type=jnp.float32)
    h += jnp.dot(t2, w2_ref[...], preferred_element_type=jnp.float32)
    h = jnp.where(h >= 0, h, 0.2 * h)
    denom = jnp.sqrt(jnp.sum(h * h, axis=1, keepdims=True))
    out_ref[...] = h / jnp.maximum(denom, 1e-12)


def _finish(sa, sb, ego_p, norm_p, w1, w2, n, block=2000):
    d = ego_p.shape[1]
    grid = (n + block - 1) // block
    return pl.pallas_call(
        _finish_kernel,
        grid=(grid,),
        in_specs=[
            pl.BlockSpec((block, d), lambda i: (i, 0)),
            pl.BlockSpec((block, d), lambda i: (i, 0)),
            pl.BlockSpec((block, d), lambda i: (i, 0)),
            pl.BlockSpec((block, 1), lambda i: (i, 0)),
            pl.BlockSpec((d, d), lambda i: (0, 0)),
            pl.BlockSpec((d, d), lambda i: (0, 0)),
        ],
        out_specs=pl.BlockSpec((block, d), lambda i: (i, 0)),
        out_shape=jax.ShapeDtypeStruct((n, d), jnp.float32),
    )(sa, sb, ego_p, norm_p, w1, w2)


@jax.jit
def kernel(ego_embedding, edge_index, norm, W1, W2):
    n, d = ego_embedding.shape
    e = edge_index.shape[1]

    npad = -(-n // (NS * 8)) * (NS * 8)
    sunit = 16
    unit_e = NS * CHUNK * sunit
    units = -(-e // unit_e)
    u0 = max(1, min(units - 1, (units * 3 + 2) // 4))
    steps0 = u0 * sunit
    steps1 = (units - u0) * sunit
    epad = units * unit_e
    e0 = NS * steps0 * CHUNK

    src = edge_index[0].astype(jnp.int32)
    dst = edge_index[1].astype(jnp.int32)
    src = jnp.pad(src, (0, epad - e), constant_values=n)
    dst = jnp.pad(dst, (0, epad - e), constant_values=0)
    src0 = src[:e0].reshape(NS, steps0, CHUNK)
    dst0 = dst[:e0].reshape(NS, steps0, CHUNK)
    src1 = src[e0:].reshape(NS, steps1, CHUNK)
    dst1 = dst[e0:].reshape(NS, steps1, CHUNK)

    ego_p = jnp.pad(ego_embedding, ((0, npad - n), (0, 0)))
    norm_p = jnp.pad(norm, ((0, npad - n), (0, 0)))

    g = _scale(ego_p, norm_p)
    parts = _make_segsum(npad, d, steps0, steps1)(g, src0, dst0, src1, dst1)
    return _finish(parts[0], parts[1], ego_p, norm_p, W1, W2, n)


# speedup vs baseline: 1.1989x; 1.1989x over previous
import functools

import jax
import jax.numpy as jnp
from jax import lax
from jax.experimental import pallas as pl
from jax.experimental.pallas import tpu as pltpu
from jax.experimental.pallas import tpu_sc as plsc

NC = 2
NS = 16
LANES = 16
CHUNK = 128
PHASES = 2


def _scale_kernel(ego_ref, norm_ref, g_ref):
    g_ref[...] = ego_ref[...] * norm_ref[...]


def _scale(ego_p, norm_p):
    npad, d = ego_p.shape
    block = npad // 16
    grid = 16
    return pl.pallas_call(
        _scale_kernel,
        grid=(grid,),
        in_specs=[
            pl.BlockSpec((block, d), lambda i: (i, 0)),
            pl.BlockSpec((block, 1), lambda i: (i, 0)),
        ],
        out_specs=pl.BlockSpec((block, d), lambda i: (i, 0)),
        out_shape=jax.ShapeDtypeStruct((npad, d), jnp.float32),
    )(ego_p, norm_p)


def _make_segsum(npad, d, steps):
    rows_per_tile = npad // NS
    zcopies = rows_per_tile // CHUNK
    zrem = rows_per_tile % CHUNK
    hsteps = steps // PHASES
    mesh = plsc.VectorSubcoreMesh(core_axis_name="c", subcore_axis_name="s")

    @functools.partial(
        pl.kernel,
        out_type=jax.ShapeDtypeStruct((NC, npad, d), jnp.float32),
        mesh=mesh,
        scratch_types=[
            pltpu.VMEM((hsteps, CHUNK), jnp.int32),
            pltpu.VMEM((hsteps, CHUNK), jnp.int32),
            pltpu.VMEM((2, CHUNK, d), jnp.float32),
            pltpu.VMEM_SHARED((npad, d), jnp.float32),
            pltpu.SemaphoreType.DMA,
            pltpu.SemaphoreType.DMA,
        ],
    )
    def segsum(g_hbm, src_hbm, dst_hbm, out_hbm, src_all, dst_all, rows, acc_sh,
               sem0, sem1):
        c = lax.axis_index("c")
        s = lax.axis_index("s")
        sems = (sem0, sem1)

        def zrow(i, _):
            def zcol(j, _):
                rows[0, i, pl.ds(j * LANES, LANES)] = jnp.zeros((LANES,), jnp.float32)
                return 0
            return lax.fori_loop(0, d // LANES, zcol, 0)
        lax.fori_loop(0, CHUNK, zrow, 0)

        zbase = s * rows_per_tile
        for k in range(zcopies):
            pltpu.sync_copy(rows.at[0], acc_sh.at[pl.ds(zbase + k * CHUNK, CHUNK)])
        if zrem:
            pltpu.sync_copy(
                rows.at[0, pl.ds(0, zrem)],
                acc_sh.at[pl.ds(zbase + zcopies * CHUNK, zrem)],
            )

        wid = s * NC + c
        plsc.subcore_barrier()

        def gather(t, b):
            pltpu.async_copy(g_hbm.at[src_all.at[t]], rows.at[b], sems[b])

        def wait_gather(t, b):
            pltpu.make_async_copy(g_hbm.at[src_all.at[t]], rows.at[b], sems[b]).wait()

        def scat(t, b):
            pltpu.sync_copy(rows.at[b], acc_sh.at[dst_all.at[t]], add=True)

        def pair(p, _):
            for b in range(2):
                t = 2 * p + b
                wait_gather(t, b)
                scat(t, b)
                gather(t + 2, b)
            return 0

        for ph in range(PHASES):
            pltpu.sync_copy(src_hbm.at[wid, pl.ds(ph * hsteps, hsteps)], src_all)
            pltpu.sync_copy(dst_hbm.at[wid, pl.ds(ph * hsteps, hsteps)], dst_all)
            gather(0, 0)
            gather(1, 1)
            lax.fori_loop(0, hsteps // 2 - 1, pair, 0)
            for b in range(2):
                t = hsteps - 2 + b
                wait_gather(t, b)
                scat(t, b)

        plsc.subcore_barrier()
        pltpu.sync_copy(
            acc_sh.at[pl.ds(zbase, rows_per_tile)],
            out_hbm.at[c, pl.ds(zbase, rows_per_tile)],
        )

    return segsum


def _finish_kernel(sa_ref, sb_ref, ego_ref, norm_ref, w1_ref, w2_ref, out_ref):
    s = sa_ref[...] + sb_ref[...]
    ego = ego_ref[...]
    nrm = norm_ref[...]
    t1 = ego + nrm * s
    t2 = (nrm * ego) * s
    h = jnp.dot(t1, w1_ref[...], preferred_element_type=jnp.float32)
    h += jnp.dot(t2, w2_ref[...], preferred_element_type=jnp.float32)
    h = jnp.where(h >= 0, h, 0.2 * h)
    denom = jnp.sqrt(jnp.sum(h * h, axis=1, keepdims=True))
    out_ref[...] = h / jnp.maximum(denom, 1e-12)


def _finish(sa, sb, ego_p, norm_p, w1, w2, n, block=2000):
    d = ego_p.shape[1]
    grid = (n + block - 1) // block
    return pl.pallas_call(
        _finish_kernel,
        grid=(grid,),
        in_specs=[
            pl.BlockSpec((block, d), lambda i: (i, 0)),
            pl.BlockSpec((block, d), lambda i: (i, 0)),
            pl.BlockSpec((block, d), lambda i: (i, 0)),
            pl.BlockSpec((block, 1), lambda i: (i, 0)),
            pl.BlockSpec((d, d), lambda i: (0, 0)),
            pl.BlockSpec((d, d), lambda i: (0, 0)),
        ],
        out_specs=pl.BlockSpec((block, d), lambda i: (i, 0)),
        out_shape=jax.ShapeDtypeStruct((n, d), jnp.float32),
    )(sa, sb, ego_p, norm_p, w1, w2)


@jax.jit
def kernel(ego_embedding, edge_index, norm, W1, W2):
    n, d = ego_embedding.shape
    e = edge_index.shape[1]

    npad = -(-n // (NS * 8)) * (NS * 8)
    nw = NC * NS
    per_w = -(-e // (nw * PHASES * 2 * CHUNK)) * (PHASES * 2 * CHUNK)
    steps = per_w // CHUNK
    epad = per_w * nw

    src = edge_index[0].astype(jnp.int32)
    dst = edge_index[1].astype(jnp.int32)
    src = jnp.pad(src, (0, epad - e), constant_values=n).reshape(nw, steps, CHUNK)
    dst = jnp.pad(dst, (0, epad - e), constant_values=0).reshape(nw, steps, CHUNK)

    ego_p = jnp.pad(ego_embedding, ((0, npad - n), (0, 0)))
    norm_p = jnp.pad(norm, ((0, npad - n), (0, 0)))

    g = _scale(ego_p, norm_p)
    parts = _make_segsum(npad, d, steps)(g, src, dst)
    return _finish(parts[0], parts[1], ego_p, norm_p, W1, W2, n)


# speedup vs baseline: 1.4825x; 1.2365x over previous
import functools

import jax
import jax.numpy as jnp
from jax import lax
from jax.experimental import pallas as pl
from jax.experimental.pallas import tpu as pltpu
from jax.experimental.pallas import tpu_sc as plsc

NC = 2
NS = 16
LANES = 16
CHUNK = 64
PHASES = 4


def _scale_kernel(ego_ref, norm_ref, gpk_ref):
    g = ego_ref[...] * norm_ref[...]
    dh = g.shape[1] // 2

    def bf16_bits(x):
        u = jax.lax.bitcast_convert_type(x, jnp.int32)
        rnd = jax.lax.shift_right_logical(u, 16) & jnp.int32(1)
        return jax.lax.shift_right_logical(u + jnp.int32(0x7FFF) + rnd, 16)

    lo = bf16_bits(g[:, :dh]) & jnp.int32(0xFFFF)
    hi = jax.lax.shift_left(bf16_bits(g[:, dh:]), 16)
    gpk_ref[...] = jax.lax.bitcast_convert_type(lo | hi, jnp.float32)


def _scale(ego_p, norm_p):
    npad, d = ego_p.shape
    block = npad // 16
    grid = 16
    return pl.pallas_call(
        _scale_kernel,
        grid=(grid,),
        in_specs=[
            pl.BlockSpec((block, d), lambda i: (i, 0)),
            pl.BlockSpec((block, 1), lambda i: (i, 0)),
        ],
        out_specs=pl.BlockSpec((block, d // 2), lambda i: (i, 0)),
        out_shape=jax.ShapeDtypeStruct((npad, d // 2), jnp.float32),
    )(ego_p, norm_p)


def _make_segsum(npad, d, steps):
    rows_per_tile = npad // NS
    zcopies = rows_per_tile // CHUNK
    zrem = rows_per_tile % CHUNK
    hsteps = steps // PHASES
    mesh = plsc.VectorSubcoreMesh(core_axis_name="c", subcore_axis_name="s")

    dh = d // 2

    @functools.partial(
        pl.kernel,
        out_type=jax.ShapeDtypeStruct((NC, npad, d), jnp.float32),
        mesh=mesh,
        compiler_params=pltpu.CompilerParams(use_tc_tiling_on_sc=False,
                                             needs_layout_passes=False),
        scratch_types=[
            pltpu.VMEM((hsteps, CHUNK), jnp.int32),
            pltpu.VMEM((hsteps, CHUNK), jnp.int32),
            pltpu.VMEM((2, CHUNK, dh), jnp.float32),
            pltpu.VMEM((2, CHUNK, d), jnp.float32),
            pltpu.VMEM_SHARED((npad, d), jnp.float32),
            pltpu.SemaphoreType.DMA,
            pltpu.SemaphoreType.DMA,
            pltpu.SemaphoreType.DMA,
            pltpu.SemaphoreType.DMA,
        ],
    )
    def segsum(gpk_hbm, src_hbm, dst_hbm, out_hbm, src_all, dst_all, pk,
               rows, acc_sh, gsem0, gsem1, ssem0, ssem1):
        c = lax.axis_index("c")
        s = lax.axis_index("s")
        gsems = (gsem0, gsem1)
        ssems = (ssem0, ssem1)

        def zrow(i, _):
            def zcol(j, _):
                rows[0, i, pl.ds(j * LANES, LANES)] = jnp.zeros((LANES,), jnp.float32)
                return 0
            return lax.fori_loop(0, d // LANES, zcol, 0)
        lax.fori_loop(0, CHUNK, zrow, 0)

        zbase = s * rows_per_tile
        for k in range(zcopies):
            pltpu.sync_copy(rows.at[0], acc_sh.at[pl.ds(zbase + k * CHUNK, CHUNK)])
        if zrem:
            pltpu.sync_copy(
                rows.at[0, pl.ds(0, zrem)],
                acc_sh.at[pl.ds(zbase + zcopies * CHUNK, zrem)],
            )

        wid = s * NC + c
        plsc.subcore_barrier()

        def gather(t, b):
            pltpu.async_copy(gpk_hbm.at[src_all.at[t]], pk.at[b], gsems[b])

        def wait_gather(t, b):
            pltpu.make_async_copy(gpk_hbm.at[src_all.at[t]], pk.at[b],
                                  gsems[b]).wait()

        def convert(b):
            def crow(i, _):
                for m in range(dh // LANES):
                    x = pk[b, i, pl.ds(m * LANES, LANES)]
                    xb = plsc.bitcast(x, jnp.bfloat16)
                    lo, hi = plsc.unpack(xb, format=plsc.PackFormat.INTERLEAVED)
                    rows[b, i, pl.ds(m * LANES, LANES)] = lo
                    rows[b, i, pl.ds(dh + m * LANES, LANES)] = hi
                return 0
            lax.fori_loop(0, CHUNK, crow, 0)

        def scat_start(t, b):
            pltpu.async_copy(rows.at[b], acc_sh.at[dst_all.at[t]], ssems[b],
                             add=True)

        def scat_wait(t, b):
            pltpu.make_async_copy(rows.at[b], acc_sh.at[dst_all.at[t]],
                                  ssems[b]).wait()

        def slot(t, b, wait_prev, do_gather):
            wait_gather(t, b)
            if wait_prev:
                scat_wait(t - 2, b)
            convert(b)
            scat_start(t, b)
            if do_gather:
                gather(t + 2, b)

        def mid(p, _):
            for b in range(2):
                t = 2 * p + b
                wait_gather(t, b)
                scat_wait(t - 2, b)
                convert(b)
                scat_start(t, b)
                gather(t + 2, b)
            return 0

        for ph in range(PHASES):
            pltpu.sync_copy(src_hbm.at[wid, pl.ds(ph * hsteps, hsteps)], src_all)
            pltpu.sync_copy(dst_hbm.at[wid, pl.ds(ph * hsteps, hsteps)], dst_all)
            gather(0, 0)
            gather(1, 1)
            slot(0, 0, wait_prev=False, do_gather=True)
            slot(1, 1, wait_prev=False, do_gather=True)
            lax.fori_loop(1, hsteps // 2 - 1, mid, 0)
            slot(hsteps - 2, 0, wait_prev=True, do_gather=False)
            slot(hsteps - 1, 1, wait_prev=True, do_gather=False)
            scat_wait(hsteps - 2, 0)
            scat_wait(hsteps - 1, 1)

        plsc.subcore_barrier()
        pltpu.sync_copy(
            acc_sh.at[pl.ds(zbase, rows_per_tile)],
            out_hbm.at[c, pl.ds(zbase, rows_per_tile)],
        )

    return segsum


def _finish_kernel(sa_ref, sb_ref, ego_ref, norm_ref, w1_ref, w2_ref, out_ref):
    s = sa_ref[...] + sb_ref[...]
    ego = ego_ref[...]
    nrm = norm_ref[...]
    t1 = ego + nrm * s
    t2 = (nrm * ego) * s
    h = jnp.dot(t1, w1_ref[...], preferred_element_type=jnp.float32)
    h += jnp.dot(t2, w2_ref[...], preferred_element_type=jnp.float32)
    h = jnp.where(h >= 0, h, 0.2 * h)
    denom = jnp.sqrt(jnp.sum(h * h, axis=1, keepdims=True))
    out_ref[...] = h / jnp.maximum(denom, 1e-12)


def _finish(sa, sb, ego_p, norm_p, w1, w2, n, block=2000):
    d = ego_p.shape[1]
    grid = (n + block - 1) // block
    return pl.pallas_call(
        _finish_kernel,
        grid=(grid,),
        in_specs=[
            pl.BlockSpec((block, d), lambda i: (i, 0)),
            pl.BlockSpec((block, d), lambda i: (i, 0)),
            pl.BlockSpec((block, d), lambda i: (i, 0)),
            pl.BlockSpec((block, 1), lambda i: (i, 0)),
            pl.BlockSpec((d, d), lambda i: (0, 0)),
            pl.BlockSpec((d, d), lambda i: (0, 0)),
        ],
        out_specs=pl.BlockSpec((block, d), lambda i: (i, 0)),
        out_shape=jax.ShapeDtypeStruct((n, d), jnp.float32),
    )(sa, sb, ego_p, norm_p, w1, w2)


@jax.jit
def kernel(ego_embedding, edge_index, norm, W1, W2):
    n, d = ego_embedding.shape
    e = edge_index.shape[1]

    npad = -(-n // (NS * 8)) * (NS * 8)
    nw = NC * NS
    per_w = -(-e // (nw * PHASES * 2 * CHUNK)) * (PHASES * 2 * CHUNK)
    steps = per_w // CHUNK
    epad = per_w * nw

    src = edge_index[0].astype(jnp.int32)
    dst = edge_index[1].astype(jnp.int32)
    src = jnp.pad(src, (0, epad - e), constant_values=n).reshape(nw, steps, CHUNK)
    dst = jnp.pad(dst, (0, epad - e), constant_values=0).reshape(nw, steps, CHUNK)

    ego_p = jnp.pad(ego_embedding, ((0, npad - n), (0, 0)))
    norm_p = jnp.pad(norm, ((0, npad - n), (0, 0)))

    g = _scale(ego_p, norm_p)
    parts = _make_segsum(npad, d, steps)(g, src, dst)
    return _finish(parts[0], parts[1], ego_p, norm_p, W1, W2, n)


# speedup vs baseline: 1.6470x; 1.1110x over previous
import functools

import jax
import jax.numpy as jnp
from jax import lax
from jax.experimental import pallas as pl
from jax.experimental.pallas import tpu as pltpu
from jax.experimental.pallas import tpu_sc as plsc

NC = 2
NS = 16
LANES = 16
CHUNK = 64
PHASES = 4


def _scale_kernel(ego_ref, norm_ref, gpk_ref):
    g = ego_ref[...] * norm_ref[...]
    dh = g.shape[1] // 2

    def bf16_bits(x):
        u = jax.lax.bitcast_convert_type(x, jnp.int32)
        rnd = jax.lax.shift_right_logical(u, 16) & jnp.int32(1)
        return jax.lax.shift_right_logical(u + jnp.int32(0x7FFF) + rnd, 16)

    lo = bf16_bits(g[:, :dh]) & jnp.int32(0xFFFF)
    hi = jax.lax.shift_left(bf16_bits(g[:, dh:]), 16)
    gpk_ref[...] = jax.lax.bitcast_convert_type(lo | hi, jnp.float32)


def _scale(ego_p, norm_p):
    npad, d = ego_p.shape
    block = npad // 16
    grid = 16
    return pl.pallas_call(
        _scale_kernel,
        grid=(grid,),
        in_specs=[
            pl.BlockSpec((block, d), lambda i: (i, 0)),
            pl.BlockSpec((block, 1), lambda i: (i, 0)),
        ],
        out_specs=pl.BlockSpec((block, d // 2), lambda i: (i, 0)),
        out_shape=jax.ShapeDtypeStruct((npad, d // 2), jnp.float32),
    )(ego_p, norm_p)


def _make_segsum(npad, d, steps):
    rows_per_tile = npad // NS
    zcopies = rows_per_tile // CHUNK
    zrem = rows_per_tile % CHUNK
    hsteps = steps // PHASES
    mesh = plsc.VectorSubcoreMesh(core_axis_name="c", subcore_axis_name="s")

    dh = d // 2

    @functools.partial(
        pl.kernel,
        out_type=jax.ShapeDtypeStruct((NC, npad, d), jnp.float32),
        mesh=mesh,
        compiler_params=pltpu.CompilerParams(use_tc_tiling_on_sc=False,
                                             needs_layout_passes=False),
        scratch_types=[
            pltpu.VMEM((hsteps, CHUNK), jnp.int32),
            pltpu.VMEM((hsteps, CHUNK), jnp.int32),
            pltpu.VMEM((2, CHUNK, dh), jnp.float32),
            pltpu.VMEM((2, CHUNK, d), jnp.float32),
            pltpu.VMEM_SHARED((npad, d), jnp.float32),
            pltpu.SemaphoreType.DMA,
            pltpu.SemaphoreType.DMA,
            pltpu.SemaphoreType.DMA,
            pltpu.SemaphoreType.DMA,
        ],
    )
    def segsum(gpk_hbm, ei_hbm, out_hbm, src_all, dst_all, pk,
               rows, acc_sh, gsem0, gsem1, ssem0, ssem1):
        c = lax.axis_index("c")
        s = lax.axis_index("s")
        gsems = (gsem0, gsem1)
        ssems = (ssem0, ssem1)

        def zrow(i, _):
            def zcol(j, _):
                rows[0, i, pl.ds(j * LANES, LANES)] = jnp.zeros((LANES,), jnp.float32)
                return 0
            return lax.fori_loop(0, d // LANES, zcol, 0)
        lax.fori_loop(0, CHUNK, zrow, 0)

        zbase = s * rows_per_tile
        for k in range(zcopies):
            pltpu.sync_copy(rows.at[0], acc_sh.at[pl.ds(zbase + k * CHUNK, CHUNK)])
        if zrem:
            pltpu.sync_copy(
                rows.at[0, pl.ds(0, zrem)],
                acc_sh.at[pl.ds(zbase + zcopies * CHUNK, zrem)],
            )

        wid = s * NC + c
        plsc.subcore_barrier()

        def gather(t, b):
            pltpu.async_copy(gpk_hbm.at[src_all.at[t]], pk.at[b], gsems[b])

        def wait_gather(t, b):
            pltpu.make_async_copy(gpk_hbm.at[src_all.at[t]], pk.at[b],
                                  gsems[b]).wait()

        def convert(b):
            def crow(i, _):
                for m in range(dh // LANES):
                    x = pk[b, i, pl.ds(m * LANES, LANES)]
                    xi = plsc.bitcast(x, jnp.int32)
                    lo = plsc.bitcast(jax.lax.shift_left(xi, 16), jnp.float32)
                    hi = plsc.bitcast(xi & jnp.int32(-65536), jnp.float32)
                    rows[b, i, pl.ds(m * LANES, LANES)] = lo
                    rows[b, i, pl.ds(dh + m * LANES, LANES)] = hi
                return 0
            lax.fori_loop(0, CHUNK, crow, 0)

        def scat_start(t, b):
            pltpu.async_copy(rows.at[b], acc_sh.at[dst_all.at[t]], ssems[b],
                             add=True)

        def scat_wait(t, b):
            pltpu.make_async_copy(rows.at[b], acc_sh.at[dst_all.at[t]],
                                  ssems[b]).wait()

        def slot(t, b, wait_prev, do_gather):
            wait_gather(t, b)
            if wait_prev:
                scat_wait(t - 2, b)
            convert(b)
            scat_start(t, b)
            if do_gather:
                gather(t + 2, b)

        def mid(p, _):
            for b in range(2):
                t = 2 * p + b
                wait_gather(t, b)
                scat_wait(t - 2, b)
                convert(b)
                scat_start(t, b)
                gather(t + 2, b)
            return 0

        for ph in range(PHASES):
            pltpu.sync_copy(ei_hbm.at[0, wid, pl.ds(ph * hsteps, hsteps)], src_all)
            pltpu.sync_copy(ei_hbm.at[1, wid, pl.ds(ph * hsteps, hsteps)], dst_all)
            gather(0, 0)
            gather(1, 1)
            slot(0, 0, wait_prev=False, do_gather=True)
            slot(1, 1, wait_prev=False, do_gather=True)
            lax.fori_loop(1, hsteps // 2 - 1, mid, 0)
            slot(hsteps - 2, 0, wait_prev=True, do_gather=False)
            slot(hsteps - 1, 1, wait_prev=True, do_gather=False)
            scat_wait(hsteps - 2, 0)
            scat_wait(hsteps - 1, 1)

        plsc.subcore_barrier()
        pltpu.sync_copy(
            acc_sh.at[pl.ds(zbase, rows_per_tile)],
            out_hbm.at[c, pl.ds(zbase, rows_per_tile)],
        )

    return segsum


def _finish_kernel(sa_ref, sb_ref, ego_ref, norm_ref, w1_ref, w2_ref, out_ref):
    s = sa_ref[...] + sb_ref[...]
    ego = ego_ref[...]
    nrm = norm_ref[...]
    t1 = ego + nrm * s
    t2 = (nrm * ego) * s
    h = jnp.dot(t1, w1_ref[...], preferred_element_type=jnp.float32)
    h += jnp.dot(t2, w2_ref[...], preferred_element_type=jnp.float32)
    h = jnp.where(h >= 0, h, 0.2 * h)
    denom = jnp.sqrt(jnp.sum(h * h, axis=1, keepdims=True))
    out_ref[...] = h / jnp.maximum(denom, 1e-12)


def _finish(sa, sb, ego_p, norm_p, w1, w2, n, block=2000):
    d = ego_p.shape[1]
    grid = (n + block - 1) // block
    return pl.pallas_call(
        _finish_kernel,
        grid=(grid,),
        in_specs=[
            pl.BlockSpec((block, d), lambda i: (i, 0)),
            pl.BlockSpec((block, d), lambda i: (i, 0)),
            pl.BlockSpec((block, d), lambda i: (i, 0)),
            pl.BlockSpec((block, 1), lambda i: (i, 0)),
            pl.BlockSpec((d, d), lambda i: (0, 0)),
            pl.BlockSpec((d, d), lambda i: (0, 0)),
        ],
        out_specs=pl.BlockSpec((block, d), lambda i: (i, 0)),
        out_shape=jax.ShapeDtypeStruct((n, d), jnp.float32),
    )(sa, sb, ego_p, norm_p, w1, w2)


@jax.jit
def kernel(ego_embedding, edge_index, norm, W1, W2):
    n, d = ego_embedding.shape
    e = edge_index.shape[1]

    npad = -(-n // (NS * 8)) * (NS * 8)
    nw = NC * NS
    per_w = -(-e // (nw * PHASES * 2 * CHUNK)) * (PHASES * 2 * CHUNK)
    steps = per_w // CHUNK
    epad = per_w * nw

    ei = jnp.pad(edge_index.astype(jnp.int32), ((0, 0), (0, epad - e)),
                 constant_values=n).reshape(2, nw, steps, CHUNK)

    ego_p = jnp.pad(ego_embedding, ((0, npad - n), (0, 0)))
    norm_p = jnp.pad(norm, ((0, npad - n), (0, 0)))

    g = _scale(ego_p, norm_p)
    parts = _make_segsum(npad, d, steps)(g, ei)
    return _finish(parts[0], parts[1], ego_p, norm_p, W1, W2, n)


# speedup vs baseline: 1.6665x; 1.0118x over previous
import functools

import jax
import jax.numpy as jnp
from jax import lax
from jax.experimental import pallas as pl
from jax.experimental.pallas import tpu as pltpu
from jax.experimental.pallas import tpu_sc as plsc

NC = 2
NS = 16
LANES = 16
CHUNK = 64
PHASES = 4


def _scale_kernel(ego_ref, norm_ref, gpk_ref):
    g = ego_ref[...] * norm_ref[...]
    dh = g.shape[1] // 2

    def bf16_bits(x):
        u = jax.lax.bitcast_convert_type(x, jnp.int32)
        rnd = jax.lax.shift_right_logical(u, 16) & jnp.int32(1)
        return jax.lax.shift_right_logical(u + jnp.int32(0x7FFF) + rnd, 16)

    lo = bf16_bits(g[:, :dh]) & jnp.int32(0xFFFF)
    hi = jax.lax.shift_left(bf16_bits(g[:, dh:]), 16)
    gpk_ref[...] = jax.lax.bitcast_convert_type(lo | hi, jnp.float32)


def _scale(ego, norm, npad, block=400):
    n, d = ego.shape
    grid = n // block
    return pl.pallas_call(
        _scale_kernel,
        grid=(grid,),
        in_specs=[
            pl.BlockSpec((block, d), lambda i: (i, 0)),
            pl.BlockSpec((block, 1), lambda i: (i, 0)),
        ],
        out_specs=pl.BlockSpec((block, d // 2), lambda i: (i, 0)),
        out_shape=jax.ShapeDtypeStruct((npad, d // 2), jnp.float32),
    )(ego, norm)


def _make_segsum(npad, d, steps):
    rows_per_tile = npad // NS
    zcopies = rows_per_tile // CHUNK
    zrem = rows_per_tile % CHUNK
    hsteps = steps // PHASES
    mesh = plsc.VectorSubcoreMesh(core_axis_name="c", subcore_axis_name="s")

    dh = d // 2

    @functools.partial(
        pl.kernel,
        out_type=jax.ShapeDtypeStruct((NC, npad, d), jnp.float32),
        mesh=mesh,
        compiler_params=pltpu.CompilerParams(use_tc_tiling_on_sc=False,
                                             needs_layout_passes=False),
        scratch_types=[
            pltpu.VMEM((hsteps, CHUNK), jnp.int32),
            pltpu.VMEM((hsteps, CHUNK), jnp.int32),
            pltpu.VMEM((2, CHUNK, dh), jnp.float32),
            pltpu.VMEM((2, CHUNK, d), jnp.float32),
            pltpu.VMEM_SHARED((npad, d), jnp.float32),
            pltpu.SemaphoreType.DMA,
            pltpu.SemaphoreType.DMA,
            pltpu.SemaphoreType.DMA,
            pltpu.SemaphoreType.DMA,
        ],
    )
    def segsum(gpk_hbm, ei_hbm, out_hbm, src_all, dst_all, pk,
               rows, acc_sh, gsem0, gsem1, ssem0, ssem1):
        c = lax.axis_index("c")
        s = lax.axis_index("s")
        gsems = (gsem0, gsem1)
        ssems = (ssem0, ssem1)

        def zrow(i, _):
            def zcol(j, _):
                rows[0, i, pl.ds(j * LANES, LANES)] = jnp.zeros((LANES,), jnp.float32)
                return 0
            return lax.fori_loop(0, d // LANES, zcol, 0)
        lax.fori_loop(0, CHUNK, zrow, 0)

        zbase = s * rows_per_tile
        for k in range(zcopies):
            pltpu.sync_copy(rows.at[0], acc_sh.at[pl.ds(zbase + k * CHUNK, CHUNK)])
        if zrem:
            pltpu.sync_copy(
                rows.at[0, pl.ds(0, zrem)],
                acc_sh.at[pl.ds(zbase + zcopies * CHUNK, zrem)],
            )

        wid = s * NC + c
        plsc.subcore_barrier()

        def gather(t, b):
            pltpu.async_copy(gpk_hbm.at[src_all.at[t]], pk.at[b], gsems[b])

        def wait_gather(t, b):
            pltpu.make_async_copy(gpk_hbm.at[src_all.at[t]], pk.at[b],
                                  gsems[b]).wait()

        def convert(b):
            def crow(i, _):
                for m in range(dh // LANES):
                    x = pk[b, i, pl.ds(m * LANES, LANES)]
                    xi = plsc.bitcast(x, jnp.int32)
                    lo = plsc.bitcast(jax.lax.shift_left(xi, 16), jnp.float32)
                    hi = plsc.bitcast(xi & jnp.int32(-65536), jnp.float32)
                    rows[b, i, pl.ds(m * LANES, LANES)] = lo
                    rows[b, i, pl.ds(dh + m * LANES, LANES)] = hi
                return 0
            lax.fori_loop(0, CHUNK, crow, 0)

        def scat_start(t, b):
            pltpu.async_copy(rows.at[b], acc_sh.at[dst_all.at[t]], ssems[b],
                             add=True)

        def scat_wait(t, b):
            pltpu.make_async_copy(rows.at[b], acc_sh.at[dst_all.at[t]],
                                  ssems[b]).wait()

        def slot(t, b, wait_prev, do_gather):
            wait_gather(t, b)
            if wait_prev:
                scat_wait(t - 2, b)
            convert(b)
            scat_start(t, b)
            if do_gather:
                gather(t + 2, b)

        def mid(p, _):
            for b in range(2):
                t = 2 * p + b
                wait_gather(t, b)
                scat_wait(t - 2, b)
                convert(b)
                scat_start(t, b)
                gather(t + 2, b)
            return 0

        for ph in range(PHASES):
            pltpu.sync_copy(ei_hbm.at[0, wid, pl.ds(ph * hsteps, hsteps)], src_all)
            pltpu.sync_copy(ei_hbm.at[1, wid, pl.ds(ph * hsteps, hsteps)], dst_all)
            gather(0, 0)
            gather(1, 1)
            slot(0, 0, wait_prev=False, do_gather=True)
            slot(1, 1, wait_prev=False, do_gather=True)
            lax.fori_loop(1, hsteps // 2 - 1, mid, 0)
            slot(hsteps - 2, 0, wait_prev=True, do_gather=False)
            slot(hsteps - 1, 1, wait_prev=True, do_gather=False)
            scat_wait(hsteps - 2, 0)
            scat_wait(hsteps - 1, 1)

        plsc.subcore_barrier()
        pltpu.sync_copy(
            acc_sh.at[pl.ds(zbase, rows_per_tile)],
            out_hbm.at[c, pl.ds(zbase, rows_per_tile)],
        )

    return segsum


def _finish_kernel(sa_ref, sb_ref, ego_ref, norm_ref, w1_ref, w2_ref, out_ref):
    s = sa_ref[0] + sb_ref[0]
    ego = ego_ref[...]
    nrm = norm_ref[...]
    t1 = ego + nrm * s
    t2 = (nrm * ego) * s
    h = jnp.dot(t1, w1_ref[...], preferred_element_type=jnp.float32)
    h += jnp.dot(t2, w2_ref[...], preferred_element_type=jnp.float32)
    h = jnp.where(h >= 0, h, 0.2 * h)
    denom = jnp.sqrt(jnp.sum(h * h, axis=1, keepdims=True))
    out_ref[...] = h / jnp.maximum(denom, 1e-12)


def _finish(parts, ego, norm, w1, w2, n, block=2000):
    d = ego.shape[1]
    grid = (n + block - 1) // block
    return pl.pallas_call(
        _finish_kernel,
        grid=(grid,),
        in_specs=[
            pl.BlockSpec((1, block, d), lambda i: (0, i, 0)),
            pl.BlockSpec((1, block, d), lambda i: (1, i, 0)),
            pl.BlockSpec((block, d), lambda i: (i, 0)),
            pl.BlockSpec((block, 1), lambda i: (i, 0)),
            pl.BlockSpec((d, d), lambda i: (0, 0)),
            pl.BlockSpec((d, d), lambda i: (0, 0)),
        ],
        out_specs=pl.BlockSpec((block, d), lambda i: (i, 0)),
        out_shape=jax.ShapeDtypeStruct((n, d), jnp.float32),
    )(parts, parts, ego, norm, w1, w2)


@jax.jit
def kernel(ego_embedding, edge_index, norm, W1, W2):
    n, d = ego_embedding.shape
    e = edge_index.shape[1]

    npad = -(-n // (NS * 8)) * (NS * 8)
    nw = NC * NS
    per_w = -(-e // (nw * PHASES * 2 * CHUNK)) * (PHASES * 2 * CHUNK)
    steps = per_w // CHUNK
    epad = per_w * nw

    ei = jnp.pad(edge_index.astype(jnp.int32), ((0, 0), (0, epad - e)),
                 constant_values=n).reshape(2, nw, steps, CHUNK)

    g = _scale(ego_embedding, norm, npad)
    parts = _make_segsum(npad, d, steps)(g, ei)
    return _finish(parts, ego_embedding, norm, W1, W2, n)
